# baseline (speedup 1.0000x reference)
import functools

import jax
import jax.numpy as jnp
from jax import lax
from jax.experimental import pallas as pl
from jax.experimental.pallas import tpu as pltpu
from jax.experimental.pallas import tpu_sc as plsc

_LOW = 0.0
_HIGH = 1.0

_NC, _NS = 2, 16
_NW = _NC * _NS
_SC_BATCH = 256
_DCH = 512
_FUNROLL = 10


def _sc_cat_body(xc_hbm, pw_hbm, out_hbm, pw_v, xc_v, acc_v, *, nb, n_cat, d):
    c = lax.axis_index("c")
    s = lax.axis_index("s")
    wid = s * _NC + c
    b0 = wid * nb
    for dc in range(d // _DCH):
        dcol = pl.ds(dc * _DCH, _DCH)
        pltpu.sync_copy(pw_hbm.at[:, dcol], pw_v)
        for b in range(nb):
            pltpu.sync_copy(xc_hbm.at[b0 + b, :, dcol], xc_v)

            def dv_body(dv, carry, b=b, dc=dc):
                col = pl.ds(dv * 16, 16)

                def f_body(fi, a):
                    for u in range(_FUNROLL):
                        f = fi * _FUNROLL + u
                        a = a + (xc_v[f, col] ^ pw_v[f, col])
                    return a

                a = lax.fori_loop(0, n_cat // _FUNROLL, f_body,
                                  jnp.zeros((16,), jnp.int32))
                acc_v[b, pl.ds(dc * _DCH + dv * 16, 16)] = a
                return carry

            lax.fori_loop(0, _DCH // 16, dv_body, 0)
    pltpu.sync_copy(acc_v, out_hbm.at[pl.ds(b0, nb)])


def _sc_cat_counts(xc, pw_cat):
    sb, n_cat, d = xc.shape
    nb = sb // _NW
    mesh = plsc.VectorSubcoreMesh(core_axis_name="c", subcore_axis_name="s")
    f = pl.kernel(
        functools.partial(_sc_cat_body, nb=nb, n_cat=n_cat, d=d),
        out_type=jax.ShapeDtypeStruct((sb, d), jnp.int32),
        mesh=mesh,
        scratch_types=[
            pltpu.VMEM((n_cat, _DCH), jnp.int32),
            pltpu.VMEM((n_cat, _DCH), jnp.int32),
            pltpu.VMEM((nb, d), jnp.int32),
        ],
    )
    return f(xc, pw_cat)


def _num_counts(xn_ref, pw_ref, lh_ref, *, n_cat, n_num, levels):
    xn = xn_ref[...]
    idx = jnp.clip(
        jnp.round((xn - _LOW) / (_HIGH - _LOW) * (levels - 1)), 0, levels - 1
    ).astype(jnp.int32)
    bb = xn.shape[0]
    oh = (idx[:, :, None]
          == lax.broadcasted_iota(jnp.int32, (bb, n_num, levels), 2)
          ).astype(jnp.float32)
    lh = lh_ref[...].astype(jnp.float32)
    num_hv = lax.dot_general(
        oh, lh, (((2,), (0,)), ((), ())), preferred_element_type=jnp.float32
    ).astype(jnp.int32)
    pwn = pw_ref[n_cat:, :]
    return jnp.sum(jnp.bitwise_xor(num_hv, pwn[None]), axis=1)


def _tc_body(xc_ref, xn_ref, pw_ref, lh_ref, out_ref, *, n_cat, n_num, levels, size):
    xc = xc_ref[...]
    pw = pw_ref[...]
    cat_counts = jnp.sum(jnp.bitwise_xor(xc, pw[None, :n_cat, :]), axis=1)
    num_counts = _num_counts(xn_ref, pw_ref, lh_ref,
                             n_cat=n_cat, n_num=n_num, levels=levels)
    total = cat_counts + num_counts
    out_ref[...] = (total * 2 >= size).astype(jnp.int32)


def _comb_body(catc_ref, xn_ref, pw_ref, lh_ref, out_ref, *, n_cat, n_num,
               levels, size):
    num_counts = _num_counts(xn_ref, pw_ref, lh_ref,
                             n_cat=n_cat, n_num=n_num, levels=levels)
    total = catc_ref[...] + num_counts
    out_ref[...] = (total * 2 >= size).astype(jnp.int32)


@jax.jit
def kernel(x_categorical, x_numeric, position_weight, level_hvs):
    b, n_cat, d = x_categorical.shape
    n_num = x_numeric.shape[1]
    size = position_weight.shape[0]
    levels = level_hvs.shape[0]
    sb = _SC_BATCH

    cat_sc = _sc_cat_counts(x_categorical[:sb], position_weight[:n_cat])

    bb = 8
    tc_b = b - sb
    body = functools.partial(
        _tc_body, n_cat=n_cat, n_num=n_num, levels=levels, size=size)
    out_tc = pl.pallas_call(
        body,
        grid=(tc_b // bb,),
        in_specs=[
            pl.BlockSpec((bb, n_cat, d), lambda i: (i, 0, 0)),
            pl.BlockSpec((bb, n_num), lambda i: (i, 0)),
            pl.BlockSpec((size, d), lambda i: (0, 0)),
            pl.BlockSpec((levels, d), lambda i: (0, 0)),
        ],
        out_specs=pl.BlockSpec((bb, d), lambda i: (i, 0)),
        out_shape=jax.ShapeDtypeStruct((tc_b, d), jnp.int32),
        compiler_params=pltpu.CompilerParams(
            dimension_semantics=("parallel",),
        ),
    )(x_categorical[sb:], x_numeric[sb:], position_weight, level_hvs)

    bbm = 32
    comb = functools.partial(
        _comb_body, n_cat=n_cat, n_num=n_num, levels=levels, size=size)
    out_sc = pl.pallas_call(
        comb,
        grid=(sb // bbm,),
        in_specs=[
            pl.BlockSpec((bbm, d), lambda i: (i, 0)),
            pl.BlockSpec((bbm, n_num), lambda i: (i, 0)),
            pl.BlockSpec((size, d), lambda i: (0, 0)),
            pl.BlockSpec((levels, d), lambda i: (0, 0)),
        ],
        out_specs=pl.BlockSpec((bbm, d), lambda i: (i, 0)),
        out_shape=jax.ShapeDtypeStruct((sb, d), jnp.int32),
        compiler_params=pltpu.CompilerParams(
            dimension_semantics=("parallel",),
        ),
    )(cat_sc, x_numeric[:sb], position_weight, level_hvs)

    return jnp.concatenate([out_sc, out_tc], axis=0)


# speedup vs baseline: 1.6493x; 1.6493x over previous
import functools

import jax
import jax.numpy as jnp
from jax import lax
from jax.experimental import pallas as pl
from jax.experimental.pallas import tpu as pltpu
from jax.experimental.pallas import tpu_sc as plsc

_LOW = 0.0
_HIGH = 1.0

_NC, _NS = 2, 16
_NW = _NC * _NS
_SC_BATCH = 256
_DCH = 512
_FUNROLL = 10


def _sc_cat_body(xc_hbm, pw_hbm, out_hbm, pw_v, xc_v, acc_v, *, nb, n_cat, d):
    c = lax.axis_index("c")
    s = lax.axis_index("s")
    wid = s * _NC + c
    b0 = wid * nb
    for dc in range(d // _DCH):
        dcol = pl.ds(dc * _DCH, _DCH)
        pltpu.sync_copy(pw_hbm.at[:, dcol], pw_v)
        for b in range(nb):
            pltpu.sync_copy(xc_hbm.at[b0 + b, :, dcol], xc_v)

            def dv_body(dv, carry, b=b, dc=dc):
                col = pl.ds(dv * 16, 16)

                def f_body(fi, a):
                    for u in range(_FUNROLL):
                        f = fi * _FUNROLL + u
                        a = a + (xc_v[f, col] ^ pw_v[f, col])
                    return a

                a = lax.fori_loop(0, n_cat // _FUNROLL, f_body,
                                  jnp.zeros((16,), jnp.int32))
                acc_v[b, pl.ds(dc * _DCH + dv * 16, 16)] = a
                return carry

            lax.fori_loop(0, _DCH // 16, dv_body, 0)
    pltpu.sync_copy(acc_v, out_hbm.at[pl.ds(b0, nb)])


def _sc_cat_counts(xc, pw_cat):
    sb, n_cat, d = xc.shape
    nb = sb // _NW
    mesh = plsc.VectorSubcoreMesh(core_axis_name="c", subcore_axis_name="s")
    f = pl.kernel(
        functools.partial(_sc_cat_body, nb=nb, n_cat=n_cat, d=d),
        out_type=jax.ShapeDtypeStruct((sb, d), jnp.int32),
        mesh=mesh,
        scratch_types=[
            pltpu.VMEM((n_cat, _DCH), jnp.int32),
            pltpu.VMEM((n_cat, _DCH), jnp.int32),
            pltpu.VMEM((nb, d), jnp.int32),
        ],
    )
    return f(xc, pw_cat)


def _num_counts(xn_ref, pw_ref, lh_ref, *, n_cat, n_num, levels):
    xn = xn_ref[...]
    idx = jnp.clip(
        jnp.round((xn - _LOW) / (_HIGH - _LOW) * (levels - 1)), 0, levels - 1
    ).astype(jnp.int32)
    bb = xn.shape[0]
    oh = (idx[:, :, None]
          == lax.broadcasted_iota(jnp.int32, (bb, n_num, levels), 2)
          ).astype(jnp.float32)
    lh = lh_ref[...].astype(jnp.float32)
    num_hv = lax.dot_general(
        oh, lh, (((2,), (0,)), ((), ())), preferred_element_type=jnp.float32
    ).astype(jnp.int32)
    pwn = pw_ref[n_cat:, :]
    return jnp.sum(jnp.bitwise_xor(num_hv, pwn[None]), axis=1)


def _tc_body(xc_ref, xn_ref, pw_ref, lh_ref, out_ref, *, n_cat, n_num, levels, size):
    xc = xc_ref[...]
    pw = pw_ref[...]
    cat_counts = jnp.sum(jnp.bitwise_xor(xc, pw[None, :n_cat, :]), axis=1)
    num_counts = _num_counts(xn_ref, pw_ref, lh_ref,
                             n_cat=n_cat, n_num=n_num, levels=levels)
    total = cat_counts + num_counts
    out_ref[...] = (total * 2 >= size).astype(jnp.int32)


def _comb_body(catc_ref, xn_ref, pw_ref, lh_ref, out_ref, *, n_cat, n_num,
               levels, size):
    num_counts = _num_counts(xn_ref, pw_ref, lh_ref,
                             n_cat=n_cat, n_num=n_num, levels=levels)
    total = catc_ref[...] + num_counts
    out_ref[...] = (total * 2 >= size).astype(jnp.int32)


@jax.jit
def kernel(x_categorical, x_numeric, position_weight, level_hvs):
    b, n_cat, d = x_categorical.shape
    n_num = x_numeric.shape[1]
    size = position_weight.shape[0]
    levels = level_hvs.shape[0]
    sb = 0

    cat_sc = (_sc_cat_counts(x_categorical[:sb], position_weight[:n_cat])
              if sb else None)

    bb = 16
    tc_b = b - sb
    body = functools.partial(
        _tc_body, n_cat=n_cat, n_num=n_num, levels=levels, size=size)
    out_tc = pl.pallas_call(
        body,
        grid=(tc_b // bb,),
        in_specs=[
            pl.BlockSpec((bb, n_cat, d), lambda i: (i, 0, 0)),
            pl.BlockSpec((bb, n_num), lambda i: (i, 0)),
            pl.BlockSpec((size, d), lambda i: (0, 0)),
            pl.BlockSpec((levels, d), lambda i: (0, 0)),
        ],
        out_specs=pl.BlockSpec((bb, d), lambda i: (i, 0)),
        out_shape=jax.ShapeDtypeStruct((tc_b, d), jnp.int32),
        compiler_params=pltpu.CompilerParams(
            dimension_semantics=("parallel",),
        ),
    )(x_categorical[sb:], x_numeric[sb:], position_weight, level_hvs)

    if not sb:
        return out_tc

    bbm = 32
    comb = functools.partial(
        _comb_body, n_cat=n_cat, n_num=n_num, levels=levels, size=size)
    out_sc = pl.pallas_call(
        comb,
        grid=(sb // bbm,),
        in_specs=[
            pl.BlockSpec((bbm, d), lambda i: (i, 0)),
            pl.BlockSpec((bbm, n_num), lambda i: (i, 0)),
            pl.BlockSpec((size, d), lambda i: (0, 0)),
            pl.BlockSpec((levels, d), lambda i: (0, 0)),
        ],
        out_specs=pl.BlockSpec((bbm, d), lambda i: (i, 0)),
        out_shape=jax.ShapeDtypeStruct((sb, d), jnp.int32),
        compiler_params=pltpu.CompilerParams(
            dimension_semantics=("parallel",),
        ),
    )(cat_sc, x_numeric[:sb], position_weight, level_hvs)

    return jnp.concatenate([out_sc, out_tc], axis=0)
